# baseline (speedup 1.0000x reference)
import functools

import jax
import jax.numpy as jnp
from jax import lax
from jax.experimental import pallas as pl
from jax.experimental.pallas import tpu as pltpu
from jax.experimental.pallas import tpu_sc as plsc

_NC = 2
_NS = 16
_NW = _NC * _NS


@functools.cache
def _make_kernel(nb, seq, emb):
    s_per_w = seq // _NW
    mesh = plsc.VectorSubcoreMesh(core_axis_name="c", subcore_axis_name="s")

    @functools.partial(
        pl.kernel,
        out_type=jax.ShapeDtypeStruct((nb, seq, emb), jnp.float32),
        mesh=mesh,
        scratch_types=[
            pltpu.VMEM((nb, s_per_w), jnp.int32),
            pltpu.VMEM((s_per_w, emb), jnp.float32),
            [pltpu.VMEM((s_per_w, emb), jnp.float32) for _ in range(nb)],
            pltpu.SemaphoreType.DMA,
            pltpu.SemaphoreType.DMA,
        ],
    )
    def tok_pos_kernel(tok_hbm, pos_hbm, idx_hbm, out_hbm, idx_v, pos_v, toks, gsem, osem):
        wid = lax.axis_index("s") * _NC + lax.axis_index("c")
        s_base = wid * s_per_w

        pltpu.sync_copy(idx_hbm.at[:, pl.ds(s_base, s_per_w)], idx_v)
        gathers = [
            pltpu.async_copy(tok_hbm.at[idx_v.at[b]], toks[b], gsem)
            for b in range(nb)
        ]
        pltpu.sync_copy(pos_hbm.at[pl.ds(s_base, s_per_w)], pos_v)

        writes = []
        for b in range(nb):
            gathers[b].wait()

            def body(r, carry):
                for u in range(emb // 16):
                    sl = pl.ds(u * 16, 16)
                    plsc.addupdate(toks[b].at[r, sl], pos_v[r, sl])
                return carry

            lax.fori_loop(0, s_per_w, body, 0)
            writes.append(
                pltpu.async_copy(toks[b], out_hbm.at[b, pl.ds(s_base, s_per_w)], osem)
            )
        for w in writes:
            w.wait()

    return tok_pos_kernel


@jax.jit
def kernel(in_idx, token_table, pos_table):
    nb, seq = in_idx.shape
    emb = token_table.shape[1]
    out = _make_kernel(nb, seq, emb)(
        token_table, pos_table, in_idx.astype(jnp.int32)
    )
    return out


# speedup vs baseline: 1.0276x; 1.0276x over previous
import functools

import jax
import jax.numpy as jnp
from jax import lax
from jax.experimental import pallas as pl
from jax.experimental.pallas import tpu as pltpu
from jax.experimental.pallas import tpu_sc as plsc

_NC = 2
_NS = 16
_NW = _NC * _NS


@functools.cache
def _make_kernel(nb, seq, emb):
    s_per_w = seq // _NW
    mesh = plsc.VectorSubcoreMesh(core_axis_name="c", subcore_axis_name="s")

    @functools.partial(
        pl.kernel,
        out_type=jax.ShapeDtypeStruct((nb, seq, emb), jnp.float32),
        mesh=mesh,
        scratch_types=[
            pltpu.VMEM((nb, s_per_w), jnp.int32),
            pltpu.VMEM((s_per_w, emb), jnp.float32),
            [pltpu.VMEM((s_per_w, emb), jnp.float32) for _ in range(nb)],
            pltpu.SemaphoreType.DMA,
            pltpu.SemaphoreType.DMA,
            pltpu.SemaphoreType.DMA,
        ],
    )
    def tok_pos_kernel(tok_hbm, pos_hbm, idx_hbm, out_hbm, idx_v, pos_v, toks, psem, gsem, osem):
        wid = lax.axis_index("s") * _NC + lax.axis_index("c")
        s_base = wid * s_per_w

        pos_d = pltpu.async_copy(pos_hbm.at[pl.ds(s_base, s_per_w)], pos_v, psem)
        pltpu.sync_copy(idx_hbm.at[:, pl.ds(s_base, s_per_w)], idx_v)
        gathers = [
            pltpu.async_copy(tok_hbm.at[idx_v.at[b]], toks[b], gsem)
            for b in range(nb)
        ]
        pos_d.wait()

        writes = []
        for b in range(nb):
            gathers[b].wait()

            def body(r, carry):
                for u in range(emb // 16):
                    sl = pl.ds(u * 16, 16)
                    plsc.addupdate(toks[b].at[r, sl], pos_v[r, sl])
                return carry

            lax.fori_loop(0, s_per_w, body, 0)
            writes.append(
                pltpu.async_copy(toks[b], out_hbm.at[b, pl.ds(s_base, s_per_w)], osem)
            )
        for w in writes:
            w.wait()

    return tok_pos_kernel


@jax.jit
def kernel(in_idx, token_table, pos_table):
    nb, seq = in_idx.shape
    emb = token_table.shape[1]
    out = _make_kernel(nb, seq, emb)(
        token_table, pos_table, in_idx.astype(jnp.int32)
    )
    return out
